# Initial kernel scaffold; baseline (speedup 1.0000x reference)
import jax
import jax.numpy as jnp
from jax.experimental import pallas as pl


def kernel(target_value, supports):
    raise NotImplementedError("write your pallas kernel here")


# speedup vs baseline: 13.9860x; 13.9860x over previous
import jax
import jax.numpy as jnp
from jax import lax
from jax.experimental import pallas as pl

EPS = 0.001
NS = 601
SMIN = -300.0
SMAX = 300.0

ROWS_PER_BLOCK = 1024


def _twohot_block(tv_ref, out_ref):
    x = tv_ref[...]
    tt = jnp.sign(x) * (jnp.sqrt(jnp.abs(x) + 1.0) - 1.0 + EPS * x)
    lowf = jnp.clip(jnp.floor(tt - SMIN), 0.0, float(NS - 1))
    low = lowf.astype(jnp.int32)
    up = jnp.minimum(low + 1, NS - 1)
    upf = up.astype(jnp.float32)
    p_low = ((upf + SMIN) - tt) / (upf - lowf)
    p_high = 1.0 - p_low
    col = lax.broadcasted_iota(jnp.int32, out_ref.shape, 1)
    out_ref[...] = jnp.where(
        col == up, p_high, jnp.where(col == low, p_low, 0.0)
    )


def kernel(target_value, supports):
    b, k = target_value.shape
    n = b * k
    tv = target_value.reshape(n, 1)
    r = ROWS_PER_BLOCK
    out = pl.pallas_call(
        _twohot_block,
        grid=(n // r,),
        in_specs=[pl.BlockSpec((r, 1), lambda i: (i, 0))],
        out_specs=pl.BlockSpec((r, NS), lambda i: (i, 0)),
        out_shape=jax.ShapeDtypeStruct((n, NS), jnp.float32),
    )(tv)
    return out.reshape(b, k, NS)


# speedup vs baseline: 14.4403x; 1.0325x over previous
import jax
import jax.numpy as jnp
from jax import lax
from jax.experimental import pallas as pl

EPS = 0.001
NS = 601
SMIN = -300.0
SMAX = 300.0

ROWS_PER_BLOCK = 1024


def _twohot_block(tv_ref, out_ref):
    x = tv_ref[...]
    tt = jnp.sign(x) * (jnp.sqrt(jnp.abs(x) + 1.0) - 1.0 + EPS * x)
    col = lax.broadcasted_iota(jnp.int32, out_ref.shape, 1)
    sup = col.astype(jnp.float32) + SMIN
    out_ref[...] = jnp.maximum(1.0 - jnp.abs(sup - tt), 0.0)


def kernel(target_value, supports):
    b, k = target_value.shape
    n = b * k
    tv = target_value.reshape(n, 1)
    r = ROWS_PER_BLOCK
    out = pl.pallas_call(
        _twohot_block,
        grid=(n // r,),
        in_specs=[pl.BlockSpec((r, 1), lambda i: (i, 0))],
        out_specs=pl.BlockSpec((r, NS), lambda i: (i, 0)),
        out_shape=jax.ShapeDtypeStruct((n, NS), jnp.float32),
    )(tv)
    return out.reshape(b, k, NS)


# speedup vs baseline: 23.4710x; 1.6254x over previous
import jax
import jax.numpy as jnp
from jax import lax
from jax.experimental import pallas as pl

EPS = 0.001
NS = 601
SMIN = -300.0

BATCH_PER_BLOCK = 16


def _twohot_block(tv_ref, out_ref):
    x = tv_ref[...]
    tt = jnp.sign(x) * (jnp.sqrt(jnp.abs(x) + 1.0) - 1.0 + EPS * x)
    col = lax.broadcasted_iota(jnp.int32, out_ref.shape, 2)
    sup = col.astype(jnp.float32) + SMIN
    out_ref[...] = jnp.maximum(1.0 - jnp.abs(sup - tt[:, :, None]), 0.0)


def kernel(target_value, supports):
    b, k = target_value.shape
    r = BATCH_PER_BLOCK
    return pl.pallas_call(
        _twohot_block,
        grid=(b // r,),
        in_specs=[pl.BlockSpec((r, k), lambda i: (i, 0))],
        out_specs=pl.BlockSpec((r, k, NS), lambda i: (i, 0, 0)),
        out_shape=jax.ShapeDtypeStruct((b, k, NS), jnp.float32),
    )(target_value)


# speedup vs baseline: 25.7962x; 1.0991x over previous
import jax
import jax.numpy as jnp
from jax import lax
from jax.experimental import pallas as pl

EPS = 0.001
NS = 601
SMIN = -300.0

BATCH_PER_BLOCK = 64


def _twohot_block(tv_ref, out_ref):
    x = tv_ref[...]
    tt = jnp.sign(x) * (jnp.sqrt(jnp.abs(x) + 1.0) - 1.0 + EPS * x)
    col = lax.broadcasted_iota(jnp.int32, out_ref.shape, 2)
    sup = col.astype(jnp.float32) + SMIN
    out_ref[...] = jnp.maximum(1.0 - jnp.abs(sup - tt[:, :, None]), 0.0)


def kernel(target_value, supports):
    b, k = target_value.shape
    r = BATCH_PER_BLOCK
    return pl.pallas_call(
        _twohot_block,
        grid=(b // r,),
        in_specs=[pl.BlockSpec((r, k), lambda i: (i, 0))],
        out_specs=pl.BlockSpec((r, k, NS), lambda i: (i, 0, 0)),
        out_shape=jax.ShapeDtypeStruct((b, k, NS), jnp.float32),
    )(target_value)
